# Initial kernel scaffold; baseline (speedup 1.0000x reference)
import jax
import jax.numpy as jnp
from jax.experimental import pallas as pl


def kernel(x, edge_index, w1a, b1a, w1b, b1b, eps1, bn1_gamma, bn1_beta, w2a, b2a, w2b, b2b, eps2, bn2_gamma, bn2_beta, wk, log_tau):
    raise NotImplementedError("write your pallas kernel here")


# speedup vs baseline: 3.0854x; 3.0854x over previous
import functools

import jax
import jax.numpy as jnp
from jax import lax
from jax.experimental import pallas as pl
from jax.experimental.pallas import tpu as pltpu
from jax.experimental.pallas import tpu_sc as plsc

_NC = 2
_NS = 16
_CH = 128


def _make_sc_agg(n_out, d, cpw, n_acc):
    mesh = plsc.VectorSubcoreMesh(
        core_axis_name="c", subcore_axis_name="s", num_cores=_NC,
        num_subcores=_NS)
    zrows = n_acc // _NS
    orows = n_out // _NS
    assert zrows % 16 == 0 and orows % 8 == 0 and n_acc >= n_out

    @functools.partial(
        pl.kernel,
        out_type=jax.ShapeDtypeStruct((_NC, n_out, d), jnp.float32),
        mesh=mesh,
        scratch_types=[
            pltpu.VMEM((cpw, _CH), jnp.int32),
            pltpu.VMEM((cpw, _CH), jnp.int32),
            pltpu.VMEM((_CH, d), jnp.float32),
            pltpu.VMEM((16, d), jnp.float32),
            pltpu.VMEM_SHARED((n_acc, d), jnp.float32),
            pltpu.SemaphoreType.DMA,
        ],
    )
    def agg(x_hbm, srcc_hbm, dstc_hbm, out_hbm, src_v, dst_v, rows_v, zbuf,
            acc, sem):
        cid = lax.axis_index("c")
        sid = lax.axis_index("s")
        wid = cid * _NS + sid

        zero16 = jnp.zeros((16,), jnp.float32)
        for i in range(16):
            for j in range(d // 16):
                zbuf[i, pl.ds(16 * j, 16)] = zero16

        def zbody(j, carry):
            pltpu.sync_copy(zbuf, acc.at[pl.ds(sid * zrows + 16 * j, 16)])
            return carry
        lax.fori_loop(0, zrows // 16, zbody, 0)

        pltpu.sync_copy(srcc_hbm.at[pl.ds(wid * cpw, cpw)], src_v)
        pltpu.sync_copy(dstc_hbm.at[pl.ds(wid * cpw, cpw)], dst_v)

        plsc.subcore_barrier()

        def ebody(j, carry):
            pltpu.async_copy(x_hbm.at[src_v.at[j]], rows_v, sem).wait()
            pltpu.sync_copy(rows_v, acc.at[dst_v.at[j]], add=True)
            return carry
        lax.fori_loop(0, cpw, ebody, 0)

        plsc.subcore_barrier()

        base = sid * orows
        done = 0
        while done < orows:
            sz = min(128, orows - done)
            r0 = base + done
            pltpu.sync_copy(acc.at[pl.ds(r0, sz)], rows_v.at[pl.ds(0, sz)])
            pltpu.sync_copy(rows_v.at[pl.ds(0, sz)],
                            out_hbm.at[cid, pl.ds(r0, sz)])
            done += sz

    return agg


def _dense1_body(eps_ref, x_ref, a0_ref, a1_ref, wa_ref, ba_ref, wb_ref,
                 bb_ref, g_ref, be_ref, out_ref):
    eps = eps_ref[0]
    h = (1.0 + eps) * x_ref[:] + a0_ref[:] + a1_ref[:]
    h = jnp.dot(h, wa_ref[:], preferred_element_type=jnp.float32) + ba_ref[:]
    h = jnp.maximum(h, 0.0)
    h = jnp.dot(h, wb_ref[:], preferred_element_type=jnp.float32) + bb_ref[:]
    m = jnp.mean(h, axis=0, keepdims=True)
    v = jnp.mean((h - m) * (h - m), axis=0, keepdims=True)
    h = (h - m) * lax.rsqrt(v + 1e-5) * g_ref[:] + be_ref[:]
    out_ref[:] = jnp.maximum(h, 0.0)


def _dense2_body(eps_ref, ltau_ref, x_ref, a0_ref, a1_ref, wa_ref, ba_ref,
                 wb_ref, bb_ref, g_ref, be_ref, wk_ref, s_ref, logit_ref):
    eps = eps_ref[0]
    h = (1.0 + eps) * x_ref[:] + a0_ref[:] + a1_ref[:]
    h = jnp.dot(h, wa_ref[:], preferred_element_type=jnp.float32) + ba_ref[:]
    h = jnp.maximum(h, 0.0)
    h = jnp.dot(h, wb_ref[:], preferred_element_type=jnp.float32) + bb_ref[:]
    m = jnp.mean(h, axis=0, keepdims=True)
    v = jnp.mean((h - m) * (h - m), axis=0, keepdims=True)
    h = (h - m) * lax.rsqrt(v + 1e-5) * g_ref[:] + be_ref[:]
    h = jnp.maximum(h, 0.0)
    logits = jnp.dot(h, wk_ref[:], preferred_element_type=jnp.float32)
    logit_ref[:] = logits
    inv_tau = jnp.exp(-ltau_ref[0])
    z = logits * inv_tau
    zmax = jnp.max(z, axis=-1, keepdims=True)
    ez = jnp.exp(z - zmax)
    s_ref[:] = ez / jnp.sum(ez, axis=-1, keepdims=True)


def _smem_spec():
    return pl.BlockSpec(memory_space=pltpu.SMEM)


def kernel(x, edge_index, w1a, b1a, w1b, b1b, eps1, bn1_gamma, bn1_beta,
           w2a, b2a, w2b, b2b, eps2, bn2_gamma, bn2_beta, wk, log_tau):
    n, d = x.shape
    h_dim = w1a.shape[1]
    k_dim = wk.shape[1]
    e = edge_index.shape[1]

    src = edge_index[0].astype(jnp.int32)
    dst = edge_index[1].astype(jnp.int32)

    nw = _NC * _NS
    cpw = -(-e // (nw * _CH))
    cpw = 8 * (-(-cpw // 8))
    ep = nw * _CH * cpw
    n_out = _NS * 8 * (-(-n // (_NS * 8)))
    n_acc = _NS * 16 * (-(-(n_out + 1) // (_NS * 16)))
    if ep > e:
        pad = ep - e
        src = jnp.concatenate([src, jnp.zeros((pad,), jnp.int32)])
        dst = jnp.concatenate([dst, jnp.full((pad,), n_out, jnp.int32)])
    src_c = src.reshape(nw * cpw, _CH)
    dst_c = dst.reshape(nw * cpw, _CH)

    sc_agg = _make_sc_agg(n_out, d, cpw, n_acc)

    eps1_a = eps1.reshape(1)
    eps2_a = eps2.reshape(1)
    ltau_a = log_tau.reshape(1)
    b1a_r, b1b_r = b1a.reshape(1, h_dim), b1b.reshape(1, h_dim)
    b2a_r, b2b_r = b2a.reshape(1, h_dim), b2b.reshape(1, h_dim)
    g1_r, be1_r = bn1_gamma.reshape(1, h_dim), bn1_beta.reshape(1, h_dim)
    g2_r, be2_r = bn2_gamma.reshape(1, h_dim), bn2_beta.reshape(1, h_dim)

    agg1 = sc_agg(x, src_c, dst_c)[:, :n, :]
    h1 = pl.pallas_call(
        _dense1_body,
        out_shape=jax.ShapeDtypeStruct((n, h_dim), jnp.float32),
        in_specs=[_smem_spec()] + [pl.BlockSpec(memory_space=pltpu.VMEM)] * 9,
    )(eps1_a, x, agg1[0], agg1[1], w1a, b1a_r, w1b, b1b_r, g1_r, be1_r)

    agg2 = sc_agg(h1, src_c, dst_c)[:, :n, :]
    s_out, logits = pl.pallas_call(
        _dense2_body,
        out_shape=(jax.ShapeDtypeStruct((n, k_dim), jnp.float32),
                   jax.ShapeDtypeStruct((n, k_dim), jnp.float32)),
        in_specs=[_smem_spec()] * 2
        + [pl.BlockSpec(memory_space=pltpu.VMEM)] * 10,
    )(eps2_a, ltau_a, h1, agg2[0], agg2[1], w2a, b2a_r, w2b, b2b_r, g2_r,
      be2_r, wk)

    return (s_out, logits)


# speedup vs baseline: 3.6982x; 1.1986x over previous
import functools

import jax
import jax.numpy as jnp
from jax import lax
from jax.experimental import pallas as pl
from jax.experimental.pallas import tpu as pltpu
from jax.experimental.pallas import tpu_sc as plsc

_NC = 2
_NS = 16
_CH = 128


def _make_sc_agg(n_out, d, cpw, n_acc):
    mesh = plsc.VectorSubcoreMesh(
        core_axis_name="c", subcore_axis_name="s", num_cores=_NC,
        num_subcores=_NS)
    zrows = n_acc // _NS
    orows = n_out // _NS
    assert zrows % _CH == 0 and orows % 8 == 0 and n_acc >= n_out

    gb = 16
    ng = cpw // gb
    assert cpw % gb == 0 and gb % 2 == 0

    @functools.partial(
        pl.kernel,
        out_type=jax.ShapeDtypeStruct((_NC, n_out, d), jnp.float32),
        mesh=mesh,
        scratch_types=[
            pltpu.VMEM((2, gb, _CH), jnp.int32),
            pltpu.VMEM((2, gb, _CH), jnp.int32),
            pltpu.VMEM((_CH, d), jnp.float32),
            pltpu.VMEM((_CH, d), jnp.float32),
            pltpu.VMEM_SHARED((n_acc, d), jnp.float32),
            pltpu.SemaphoreType.DMA,
            pltpu.SemaphoreType.DMA,
            pltpu.SemaphoreType.DMA,
        ],
    )
    def agg(x_hbm, srcc_hbm, dstc_hbm, out_hbm, src_v, dst_v, rows_a, rows_b,
            acc, sem_a, sem_b, sem_i):
        cid = lax.axis_index("c")
        sid = lax.axis_index("s")
        wid = cid * _NS + sid
        cbase = wid * cpw

        zero16 = jnp.zeros((16,), jnp.float32)
        for i in range(_CH):
            for j in range(d // 16):
                rows_a[i, pl.ds(16 * j, 16)] = zero16

        def zbody(j, carry):
            pltpu.sync_copy(rows_a, acc.at[pl.ds(sid * zrows + _CH * j, _CH)])
            return carry
        lax.fori_loop(0, zrows // _CH, zbody, 0)

        pltpu.sync_copy(srcc_hbm.at[pl.ds(cbase, gb)], src_v.at[0])
        pltpu.sync_copy(dstc_hbm.at[pl.ds(cbase, gb)], dst_v.at[0])

        plsc.subcore_barrier()

        def gbody(g, carry):
            slot = lax.rem(g, 2)

            @pl.when(g > 0)
            def _():
                pltpu.make_async_copy(
                    srcc_hbm.at[pl.ds(cbase, gb)], src_v.at[slot],
                    sem_i).wait()
                pltpu.make_async_copy(
                    dstc_hbm.at[pl.ds(cbase, gb)], dst_v.at[slot],
                    sem_i).wait()

            @pl.when(g + 1 < ng)
            def _():
                nxt = cbase + (g + 1) * gb
                pltpu.async_copy(
                    srcc_hbm.at[pl.ds(nxt, gb)], src_v.at[1 - slot], sem_i)
                pltpu.async_copy(
                    dstc_hbm.at[pl.ds(nxt, gb)], dst_v.at[1 - slot], sem_i)

            pltpu.async_copy(x_hbm.at[src_v.at[slot, 0]], rows_a, sem_a)

            def ebody(i, carry2):
                j = 2 * i
                pltpu.async_copy(
                    x_hbm.at[src_v.at[slot, j + 1]], rows_b, sem_b)
                pltpu.make_async_copy(
                    x_hbm.at[src_v.at[slot, j]], rows_a, sem_a).wait()
                pltpu.sync_copy(rows_a, acc.at[dst_v.at[slot, j]], add=True)

                @pl.when(j + 2 < gb)
                def _():
                    pltpu.async_copy(
                        x_hbm.at[src_v.at[slot, j + 2]], rows_a, sem_a)
                pltpu.make_async_copy(
                    x_hbm.at[src_v.at[slot, j + 1]], rows_b, sem_b).wait()
                pltpu.sync_copy(
                    rows_b, acc.at[dst_v.at[slot, j + 1]], add=True)
                return carry2
            lax.fori_loop(0, gb // 2, ebody, 0)
            return carry
        lax.fori_loop(0, ng, gbody, 0)

        plsc.subcore_barrier()

        base = sid * orows
        sizes = []
        done = 0
        while done < orows:
            sizes.append(min(_CH, orows - done))
            done += sizes[-1]
        offs = [base + sum(sizes[:i]) for i in range(len(sizes))]
        descs = []
        for i, (r0, sz) in enumerate(zip(offs, sizes)):
            buf = rows_a if i % 2 == 0 else rows_b
            sem = sem_a if i % 2 == 0 else sem_b
            if i >= 2:
                descs[i - 2].wait()
            pltpu.sync_copy(acc.at[pl.ds(r0, sz)], buf.at[pl.ds(0, sz)])
            descs.append(pltpu.async_copy(
                buf.at[pl.ds(0, sz)], out_hbm.at[cid, pl.ds(r0, sz)], sem))
        for dsc in descs[-2:]:
            dsc.wait()

    return agg


def _dense1_body(eps_ref, x_ref, agg_ref, wa_ref, ba_ref, wb_ref,
                 bb_ref, g_ref, be_ref, out_ref):
    eps = eps_ref[0]
    n = x_ref.shape[0]
    h = ((1.0 + eps) * x_ref[:] + agg_ref[0, :n, :] + agg_ref[1, :n, :])
    h = jnp.dot(h, wa_ref[:], preferred_element_type=jnp.float32) + ba_ref[:]
    h = jnp.maximum(h, 0.0)
    h = jnp.dot(h, wb_ref[:], preferred_element_type=jnp.float32) + bb_ref[:]
    m = jnp.mean(h, axis=0, keepdims=True)
    v = jnp.mean((h - m) * (h - m), axis=0, keepdims=True)
    h = (h - m) * lax.rsqrt(v + 1e-5) * g_ref[:] + be_ref[:]
    out_ref[:] = jnp.maximum(h, 0.0)


def _dense2_body(eps_ref, ltau_ref, x_ref, agg_ref, wa_ref, ba_ref,
                 wb_ref, bb_ref, g_ref, be_ref, wk_ref, s_ref, logit_ref):
    eps = eps_ref[0]
    n = x_ref.shape[0]
    h = ((1.0 + eps) * x_ref[:] + agg_ref[0, :n, :] + agg_ref[1, :n, :])
    h = jnp.dot(h, wa_ref[:], preferred_element_type=jnp.float32) + ba_ref[:]
    h = jnp.maximum(h, 0.0)
    h = jnp.dot(h, wb_ref[:], preferred_element_type=jnp.float32) + bb_ref[:]
    m = jnp.mean(h, axis=0, keepdims=True)
    v = jnp.mean((h - m) * (h - m), axis=0, keepdims=True)
    h = (h - m) * lax.rsqrt(v + 1e-5) * g_ref[:] + be_ref[:]
    h = jnp.maximum(h, 0.0)
    logits = jnp.dot(h, wk_ref[:], preferred_element_type=jnp.float32)
    logit_ref[:] = logits
    inv_tau = jnp.exp(-ltau_ref[0])
    z = logits * inv_tau
    zmax = jnp.max(z, axis=-1, keepdims=True)
    ez = jnp.exp(z - zmax)
    s_ref[:] = ez / jnp.sum(ez, axis=-1, keepdims=True)


def _smem_spec():
    return pl.BlockSpec(memory_space=pltpu.SMEM)


def kernel(x, edge_index, w1a, b1a, w1b, b1b, eps1, bn1_gamma, bn1_beta,
           w2a, b2a, w2b, b2b, eps2, bn2_gamma, bn2_beta, wk, log_tau):
    n, d = x.shape
    h_dim = w1a.shape[1]
    k_dim = wk.shape[1]
    e = edge_index.shape[1]

    src = edge_index[0].astype(jnp.int32)
    dst = edge_index[1].astype(jnp.int32)

    nw = _NC * _NS
    cpw = -(-e // (nw * _CH))
    cpw = 8 * (-(-cpw // 8))
    ep = nw * _CH * cpw
    n_out = _NS * 8 * (-(-n // (_NS * 8)))
    n_acc = _NS * 16 * (-(-(n_out + 1) // (_NS * 16)))
    if ep > e:
        pad = ep - e
        src = jnp.concatenate([src, jnp.zeros((pad,), jnp.int32)])
        dst = jnp.concatenate([dst, jnp.full((pad,), n_out, jnp.int32)])
    src_c = src.reshape(nw * cpw, _CH)
    dst_c = dst.reshape(nw * cpw, _CH)

    sc_agg = _make_sc_agg(n_out, d, cpw, n_acc)

    eps1_a = eps1.reshape(1)
    eps2_a = eps2.reshape(1)
    ltau_a = log_tau.reshape(1)
    b1a_r, b1b_r = b1a.reshape(1, h_dim), b1b.reshape(1, h_dim)
    b2a_r, b2b_r = b2a.reshape(1, h_dim), b2b.reshape(1, h_dim)
    g1_r, be1_r = bn1_gamma.reshape(1, h_dim), bn1_beta.reshape(1, h_dim)
    g2_r, be2_r = bn2_gamma.reshape(1, h_dim), bn2_beta.reshape(1, h_dim)

    agg1 = sc_agg(x, src_c, dst_c)
    h1 = pl.pallas_call(
        _dense1_body,
        out_shape=jax.ShapeDtypeStruct((n, h_dim), jnp.float32),
        in_specs=[_smem_spec()] + [pl.BlockSpec(memory_space=pltpu.VMEM)] * 8,
    )(eps1_a, x, agg1, w1a, b1a_r, w1b, b1b_r, g1_r, be1_r)

    agg2 = sc_agg(h1, src_c, dst_c)
    s_out, logits = pl.pallas_call(
        _dense2_body,
        out_shape=(jax.ShapeDtypeStruct((n, k_dim), jnp.float32),
                   jax.ShapeDtypeStruct((n, k_dim), jnp.float32)),
        in_specs=[_smem_spec()] * 2
        + [pl.BlockSpec(memory_space=pltpu.VMEM)] * 9,
    )(eps2_a, ltau_a, h1, agg2, w2a, b2a_r, w2b, b2b_r, g2_r,
      be2_r, wk)

    return (s_out, logits)
